# Initial kernel scaffold; baseline (speedup 1.0000x reference)
import functools

import jax
import jax.numpy as jnp
from jax import lax
from jax.experimental import pallas as pl
from jax.experimental.pallas import tpu as pltpu
from jax.experimental.pallas import tpu_sc as plsc

NC = 2
NS = 16
L = 16

N_USERS = 25000
N_ITEMS = 25000
N_NODES = N_USERS + N_ITEMS
D = 64
DH = D // 2
N_LAYERS = 3
BATCH_B = 4096

E = 800000
CHUNK = 128
CHUNKS_PER_TILE = 200
E_PAD = NS * CHUNKS_PER_TILE * CHUNK
ROWS_PER_TILE = N_NODES // NS

_mesh = plsc.VectorSubcoreMesh(core_axis_name="c", subcore_axis_name="s")


def _layer_body(tab32, src_h, dst_h, val_h, zeros_h, raw_h,
                acc, src_v, src2_v, dst_v, val_v, rows_v, sem):
    c = lax.axis_index("c")
    s = lax.axis_index("s")

    pltpu.sync_copy(zeros_h, acc.at[pl.ds(s * ROWS_PER_TILE, ROWS_PER_TILE)])
    plsc.subcore_barrier()

    base_edge = s * (CHUNKS_PER_TILE * CHUNK)

    def chunk_body(i, carry):
        off = base_edge + i * CHUNK
        pltpu.sync_copy(src_h.at[pl.ds(off, CHUNK)], src_v)
        pltpu.sync_copy(dst_h.at[pl.ds(off, CHUNK)], dst_v)
        pltpu.sync_copy(val_h.at[pl.ds(off, CHUNK)], val_v)

        for k in range(CHUNK // L):
            sl = pl.ds(k * L, L)
            src2_v[sl] = src_v[sl] * 2 + c

        pltpu.async_copy(tab32.at[src2_v], rows_v, sem).wait()

        for g in range(CHUNK // L):
            ev = lax.iota(jnp.int32, (L,)) + g * L
            v = val_v[pl.ds(g * L, L)]
            for d in range(DH):
                dv = jnp.zeros((L,), jnp.int32) + d
                x = plsc.load_gather(rows_v, [ev, dv])
                plsc.store_scatter(rows_v, [ev, dv], x * v)

        pltpu.sync_copy(rows_v, acc.at[dst_v], add=True)
        return carry

    lax.fori_loop(0, CHUNKS_PER_TILE, chunk_body, 0)
    plsc.subcore_barrier()

    ro = s * ROWS_PER_TILE
    pltpu.sync_copy(acc.at[pl.ds(ro, ROWS_PER_TILE)],
                    raw_h.at[pl.ds(c * N_NODES + ro, ROWS_PER_TILE)])


_layer_call = pl.kernel(
    _layer_body,
    out_type=jax.ShapeDtypeStruct((NC * N_NODES, DH), jnp.float32),
    mesh=_mesh,
    scratch_types=[
        pltpu.VMEM_SHARED((N_NODES, DH), jnp.float32),
        pltpu.VMEM((CHUNK,), jnp.int32),
        pltpu.VMEM((CHUNK,), jnp.int32),
        pltpu.VMEM((CHUNK,), jnp.int32),
        pltpu.VMEM((CHUNK,), jnp.float32),
        pltpu.VMEM((CHUNK, DH), jnp.float32),
        pltpu.SemaphoreType.DMA,
    ],
)


def _norm_body(scale, raw_ref, accin_ref, norm_ref, accout_ref):
    ra = raw_ref[0]
    rb = raw_ref[1]
    ss = (jnp.sum(ra * ra, axis=1, keepdims=True)
          + jnp.sum(rb * rb, axis=1, keepdims=True))
    r = lax.rsqrt(jnp.maximum(ss, 1e-12))
    full = jnp.concatenate([ra * r, rb * r], axis=1)
    norm_ref[...] = full
    accout_ref[...] = scale * accin_ref[...] + 0.25 * full


_NORM_ROWS = 1000


def _make_norm_call(scale):
    return pl.pallas_call(
        functools.partial(_norm_body, scale),
        grid=(N_NODES // _NORM_ROWS,),
        in_specs=[
            pl.BlockSpec((NC, _NORM_ROWS, DH), lambda i: (0, i, 0)),
            pl.BlockSpec((_NORM_ROWS, D), lambda i: (i, 0)),
        ],
        out_specs=[
            pl.BlockSpec((_NORM_ROWS, D), lambda i: (i, 0)),
            pl.BlockSpec((_NORM_ROWS, D), lambda i: (i, 0)),
        ],
        out_shape=[
            jax.ShapeDtypeStruct((N_NODES, D), jnp.float32),
            jax.ShapeDtypeStruct((N_NODES, D), jnp.float32),
        ],
    )


_norm_first = _make_norm_call(0.25)
_norm_rest = _make_norm_call(1.0)


_B_PER_TILE = BATCH_B // (NC * NS)


def _final_body(light, users_h, pos_h, neg_h, pos_out, neg_out,
                u_idx, p_idx, n_idx, u_rows, p_rows, n_rows, scores_p,
                scores_n, sem):
    c = lax.axis_index("c")
    s = lax.axis_index("s")
    wid = s * NC + c
    base = wid * _B_PER_TILE

    pltpu.sync_copy(users_h.at[pl.ds(base, _B_PER_TILE)], u_idx)
    pltpu.sync_copy(pos_h.at[pl.ds(base, _B_PER_TILE)], p_idx)
    pltpu.sync_copy(neg_h.at[pl.ds(base, _B_PER_TILE)], n_idx)

    for k in range(_B_PER_TILE // L):
        sl = pl.ds(k * L, L)
        p_idx[sl] = p_idx[sl] + N_USERS
        n_idx[sl] = n_idx[sl] + N_USERS

    pltpu.async_copy(light.at[u_idx], u_rows, sem).wait()
    pltpu.async_copy(light.at[p_idx], p_rows, sem).wait()
    pltpu.async_copy(light.at[n_idx], n_rows, sem).wait()

    for g in range(_B_PER_TILE // L):
        ev = lax.iota(jnp.int32, (L,)) + g * L
        accp = jnp.zeros((L,), jnp.float32)
        accn = jnp.zeros((L,), jnp.float32)
        for d in range(D):
            dv = jnp.zeros((L,), jnp.int32) + d
            u = plsc.load_gather(u_rows, [ev, dv])
            p = plsc.load_gather(p_rows, [ev, dv])
            n = plsc.load_gather(n_rows, [ev, dv])
            accp = accp + u * p
            accn = accn + u * n
        scores_p[pl.ds(g * L, L)] = accp
        scores_n[pl.ds(g * L, L)] = accn

    pltpu.sync_copy(scores_p, pos_out.at[pl.ds(base, _B_PER_TILE)])
    pltpu.sync_copy(scores_n, neg_out.at[pl.ds(base, _B_PER_TILE)])


_final_call = pl.kernel(
    _final_body,
    out_type=[
        jax.ShapeDtypeStruct((BATCH_B,), jnp.float32),
        jax.ShapeDtypeStruct((BATCH_B,), jnp.float32),
    ],
    mesh=_mesh,
    scratch_types=[
        pltpu.VMEM((_B_PER_TILE,), jnp.int32),
        pltpu.VMEM((_B_PER_TILE,), jnp.int32),
        pltpu.VMEM((_B_PER_TILE,), jnp.int32),
        pltpu.VMEM((_B_PER_TILE, D), jnp.float32),
        pltpu.VMEM((_B_PER_TILE, D), jnp.float32),
        pltpu.VMEM((_B_PER_TILE, D), jnp.float32),
        pltpu.VMEM((_B_PER_TILE,), jnp.float32),
        pltpu.VMEM((_B_PER_TILE,), jnp.float32),
        pltpu.SemaphoreType.DMA,
    ],
)


def kernel(users, pos_items, neg_items, adj_indices, adj_values, user_table,
           item_table):
    users = users.astype(jnp.int32)
    pos_items = pos_items.astype(jnp.int32)
    neg_items = neg_items.astype(jnp.int32)

    dst = adj_indices[0].astype(jnp.int32)
    src = adj_indices[1].astype(jnp.int32)
    vals = adj_values.astype(jnp.float32)

    pad = E_PAD - E
    src_p = jnp.concatenate([src, jnp.zeros((pad,), jnp.int32)])
    dst_p = jnp.concatenate([dst, jnp.zeros((pad,), jnp.int32)])
    val_p = jnp.concatenate([vals, jnp.zeros((pad,), jnp.float32)])

    all_emb = jnp.concatenate([user_table, item_table], axis=0)
    zeros = jnp.zeros((ROWS_PER_TILE, DH), jnp.float32)

    tab = all_emb
    acc = all_emb
    for layer in range(N_LAYERS):
        raw = _layer_call(tab.reshape(NC * N_NODES, DH), src_p, dst_p, val_p,
                          zeros)
        norm_call = _norm_first if layer == 0 else _norm_rest
        tab, acc = norm_call(raw.reshape(NC, N_NODES, DH), acc)

    pos_scores, neg_scores = _final_call(acc, users, pos_items, neg_items)
    return (pos_scores, neg_scores, acc[:N_USERS], acc[N_USERS:])


# speedup vs baseline: 3.0157x; 3.0157x over previous
import functools

import jax
import jax.numpy as jnp
from jax import lax
from jax.experimental import pallas as pl
from jax.experimental.pallas import tpu as pltpu
from jax.experimental.pallas import tpu_sc as plsc

NC = 2
NS = 16
L = 16

N_USERS = 25000
N_ITEMS = 25000
N_NODES = N_USERS + N_ITEMS
D = 64
DH = D // 2
N_LAYERS = 3
BATCH_B = 4096

E = 800000
CHUNK = 128
CHUNKS_PER_TILE = 391
E_PAD = NS * CHUNKS_PER_TILE * CHUNK
ROWS_PER_TILE = 3128
ROWS_LAST = N_NODES - (NS - 1) * ROWS_PER_TILE

_mesh = plsc.VectorSubcoreMesh(core_axis_name="c", subcore_axis_name="s")
_sc_params = pltpu.CompilerParams(use_tc_tiling_on_sc=False)


def _layer_body(tab32, src_h, dst_h, val_h, zeros_h, raw_h,
                acc, src_v, src2_v, dst_v, val_v, rows_v, sem):
    c = lax.axis_index("c")
    s = lax.axis_index("s")

    @pl.when(s < NS - 1)
    def _():
        pltpu.sync_copy(zeros_h,
                        acc.at[pl.ds(s * ROWS_PER_TILE, ROWS_PER_TILE)])

    @pl.when(s == NS - 1)
    def _():
        pltpu.sync_copy(zeros_h.at[pl.ds(0, ROWS_LAST)],
                        acc.at[pl.ds((NS - 1) * ROWS_PER_TILE, ROWS_LAST)])

    plsc.subcore_barrier()

    base_edge = s * (CHUNKS_PER_TILE * CHUNK)

    def chunk_body(i, carry):
        off = base_edge + i * CHUNK
        pltpu.sync_copy(src_h.at[pl.ds(off, CHUNK)], src_v)
        pltpu.sync_copy(dst_h.at[pl.ds(off, CHUNK)], dst_v)
        pltpu.sync_copy(val_h.at[pl.ds(off, CHUNK)], val_v)

        for k in range(CHUNK // L):
            sl = pl.ds(k * L, L)
            src2_v[sl] = src_v[sl] * 2 + c

        pltpu.async_copy(tab32.at[src2_v], rows_v, sem).wait()

        for g in range(CHUNK // L):
            vv16 = val_v[pl.ds(g * L, L)]
            for e in range(L):
                r = g * L + e
                vv = jnp.full((L,), vv16[e], jnp.float32)
                rows_v[r, pl.ds(0, L)] = rows_v[r, pl.ds(0, L)] * vv
                rows_v[r, pl.ds(L, L)] = rows_v[r, pl.ds(L, L)] * vv

        pltpu.sync_copy(rows_v, acc.at[dst_v], add=True)
        return carry

    lax.fori_loop(0, CHUNKS_PER_TILE, chunk_body, 0)
    plsc.subcore_barrier()

    ro = s * ROWS_PER_TILE

    @pl.when(s < NS - 1)
    def _():
        pltpu.sync_copy(acc.at[pl.ds(ro, ROWS_PER_TILE)],
                        raw_h.at[pl.ds(c * N_NODES + ro, ROWS_PER_TILE)])

    @pl.when(s == NS - 1)
    def _():
        ro_l = (NS - 1) * ROWS_PER_TILE
        pltpu.sync_copy(acc.at[pl.ds(ro_l, ROWS_LAST)],
                        raw_h.at[pl.ds(c * N_NODES + ro_l, ROWS_LAST)])


_layer_call = pl.kernel(
    _layer_body,
    out_type=jax.ShapeDtypeStruct((NC * N_NODES, DH), jnp.float32),
    mesh=_mesh,
    compiler_params=_sc_params,
    scratch_types=[
        pltpu.VMEM_SHARED((N_NODES, DH), jnp.float32),
        pltpu.VMEM((CHUNK,), jnp.int32),
        pltpu.VMEM((CHUNK,), jnp.int32),
        pltpu.VMEM((CHUNK,), jnp.int32),
        pltpu.VMEM((CHUNK,), jnp.float32),
        pltpu.VMEM((CHUNK, DH), jnp.float32),
        pltpu.SemaphoreType.DMA,
    ],
)


def _norm_body(scale, raw_ref, accin_ref, norm_ref, accout_ref):
    ra = raw_ref[0]
    rb = raw_ref[1]
    ss = (jnp.sum(ra * ra, axis=1, keepdims=True)
          + jnp.sum(rb * rb, axis=1, keepdims=True))
    r = lax.rsqrt(jnp.maximum(ss, 1e-12))
    full = jnp.concatenate([ra * r, rb * r], axis=1)
    norm_ref[...] = full
    accout_ref[...] = scale * accin_ref[...] + 0.25 * full


_NORM_ROWS = 1000


def _make_norm_call(scale):
    return pl.pallas_call(
        functools.partial(_norm_body, scale),
        grid=(N_NODES // _NORM_ROWS,),
        in_specs=[
            pl.BlockSpec((NC, _NORM_ROWS, DH), lambda i: (0, i, 0)),
            pl.BlockSpec((_NORM_ROWS, D), lambda i: (i, 0)),
        ],
        out_specs=[
            pl.BlockSpec((_NORM_ROWS, D), lambda i: (i, 0)),
            pl.BlockSpec((_NORM_ROWS, D), lambda i: (i, 0)),
        ],
        out_shape=[
            jax.ShapeDtypeStruct((N_NODES, D), jnp.float32),
            jax.ShapeDtypeStruct((N_NODES, D), jnp.float32),
        ],
    )


_norm_first = _make_norm_call(0.25)
_norm_rest = _make_norm_call(1.0)


_B_PER_TILE = BATCH_B // (NC * NS)


def _gather_body(light, users_h, pos_h, neg_h, u_out, p_out, n_out,
                 u_idx, p_idx, n_idx, rows_u, rows_p, rows_n, sem):
    c = lax.axis_index("c")
    s = lax.axis_index("s")
    wid = s * NC + c
    base = wid * _B_PER_TILE

    pltpu.sync_copy(users_h.at[pl.ds(base, _B_PER_TILE)], u_idx)
    pltpu.sync_copy(pos_h.at[pl.ds(base, _B_PER_TILE)], p_idx)
    pltpu.sync_copy(neg_h.at[pl.ds(base, _B_PER_TILE)], n_idx)

    for k in range(_B_PER_TILE // L):
        sl = pl.ds(k * L, L)
        p_idx[sl] = p_idx[sl] + N_USERS
        n_idx[sl] = n_idx[sl] + N_USERS

    pltpu.async_copy(light.at[u_idx], rows_u, sem).wait()
    pltpu.async_copy(light.at[p_idx], rows_p, sem).wait()
    pltpu.async_copy(light.at[n_idx], rows_n, sem).wait()

    pltpu.sync_copy(rows_u, u_out.at[pl.ds(base, _B_PER_TILE)])
    pltpu.sync_copy(rows_p, p_out.at[pl.ds(base, _B_PER_TILE)])
    pltpu.sync_copy(rows_n, n_out.at[pl.ds(base, _B_PER_TILE)])


_gather_call = pl.kernel(
    _gather_body,
    out_type=[
        jax.ShapeDtypeStruct((BATCH_B, D), jnp.float32),
        jax.ShapeDtypeStruct((BATCH_B, D), jnp.float32),
        jax.ShapeDtypeStruct((BATCH_B, D), jnp.float32),
    ],
    mesh=_mesh,
    compiler_params=_sc_params,
    scratch_types=[
        pltpu.VMEM((_B_PER_TILE,), jnp.int32),
        pltpu.VMEM((_B_PER_TILE,), jnp.int32),
        pltpu.VMEM((_B_PER_TILE,), jnp.int32),
        pltpu.VMEM((_B_PER_TILE, D), jnp.float32),
        pltpu.VMEM((_B_PER_TILE, D), jnp.float32),
        pltpu.VMEM((_B_PER_TILE, D), jnp.float32),
        pltpu.SemaphoreType.DMA,
    ],
)


def _dot_body(u_ref, p_ref, n_ref, ps_ref, ns_ref):
    u = u_ref[...]
    ps_ref[...] = jnp.sum(u * p_ref[...], axis=1)
    ns_ref[...] = jnp.sum(u * n_ref[...], axis=1)


_dot_call = pl.pallas_call(
    _dot_body,
    out_shape=[
        jax.ShapeDtypeStruct((BATCH_B,), jnp.float32),
        jax.ShapeDtypeStruct((BATCH_B,), jnp.float32),
    ],
)


def kernel(users, pos_items, neg_items, adj_indices, adj_values, user_table,
           item_table):
    users = users.astype(jnp.int32)
    pos_items = pos_items.astype(jnp.int32)
    neg_items = neg_items.astype(jnp.int32)

    dst = adj_indices[0].astype(jnp.int32)
    src = adj_indices[1].astype(jnp.int32)
    vals = adj_values.astype(jnp.float32)

    pad = E_PAD - E
    src_p = jnp.concatenate([src, jnp.zeros((pad,), jnp.int32)])
    dst_p = jnp.concatenate([dst, jnp.zeros((pad,), jnp.int32)])
    val_p = jnp.concatenate([vals, jnp.zeros((pad,), jnp.float32)])

    all_emb = jnp.concatenate([user_table, item_table], axis=0)
    zeros = jnp.zeros((ROWS_PER_TILE, DH), jnp.float32)

    tab = all_emb
    acc = all_emb
    for layer in range(N_LAYERS):
        raw = _layer_call(tab.reshape(NC * N_NODES, DH), src_p, dst_p, val_p,
                          zeros)
        norm_call = _norm_first if layer == 0 else _norm_rest
        tab, acc = norm_call(raw.reshape(NC, N_NODES, DH), acc)

    u_rows, p_rows, n_rows = _gather_call(acc, users, pos_items, neg_items)
    pos_scores, neg_scores = _dot_call(u_rows, p_rows, n_rows)
    return (pos_scores, neg_scores, acc[:N_USERS], acc[N_USERS:])


# speedup vs baseline: 4.3005x; 1.4260x over previous
import functools

import jax
import jax.numpy as jnp
from jax import lax
from jax.experimental import pallas as pl
from jax.experimental.pallas import tpu as pltpu
from jax.experimental.pallas import tpu_sc as plsc

NC = 2
NS = 16
L = 16

N_USERS = 25000
N_ITEMS = 25000
N_NODES = N_USERS + N_ITEMS
D = 64
DH = D // 2
N_LAYERS = 3
BATCH_B = 4096

E = 800000
CHUNK = 128
UNROLL = 4
N_GROUPS = 100
CHUNKS_PER_TILE = UNROLL * N_GROUPS
E_PAD = NS * CHUNKS_PER_TILE * CHUNK
E_ALLOC = E_PAD + 8 * CHUNK
ROWS_PER_TILE = 3128
ROWS_LAST = N_NODES - (NS - 1) * ROWS_PER_TILE

_mesh = plsc.VectorSubcoreMesh(core_axis_name="c", subcore_axis_name="s")
_sc_params = pltpu.CompilerParams(use_tc_tiling_on_sc=False)


def _layer_body(tab32, src_h, dst_h, val_h, zeros_h, raw_h,
                acc, src_big, dst_big, val_big,
                src2_a, src2_b, dst2_a, dst2_b, rows_a, rows_b,
                gsem_a, gsem_b, ssem_a, ssem_b, isem_a, isem_b):
    c = lax.axis_index("c")
    s = lax.axis_index("s")

    @pl.when(s < NS - 1)
    def _():
        pltpu.sync_copy(zeros_h,
                        acc.at[pl.ds(s * ROWS_PER_TILE, ROWS_PER_TILE)])

    @pl.when(s == NS - 1)
    def _():
        pltpu.sync_copy(zeros_h.at[pl.ds(0, ROWS_LAST)],
                        acc.at[pl.ds((NS - 1) * ROWS_PER_TILE, ROWS_LAST)])

    plsc.subcore_barrier()

    tile_row0 = s * CHUNKS_PER_TILE
    src2 = (src2_a, src2_b)
    dst2 = (dst2_a, dst2_b)
    rows = (rows_a, rows_b)
    gsem = (gsem_a, gsem_b)
    ssem = (ssem_a, ssem_b)
    isem = (isem_a, isem_b)

    def issue_load(pair, slot):
        hrow = tile_row0 + pair * UNROLL + slot * 2
        brow = slot * 2
        for h, b in ((src_h, src_big), (dst_h, dst_big), (val_h, val_big)):
            pltpu.async_copy(h.at[pl.ds(hrow, 2), :],
                             b.at[pl.ds(brow, 2), :], isem[slot])

    def wait_load(slot):
        brow = slot * 2
        for h, b in ((src_h, src_big), (dst_h, dst_big), (val_h, val_big)):
            pltpu.make_async_copy(h.at[pl.ds(0, 2), :],
                                  b.at[pl.ds(brow, 2), :], isem[slot]).wait()

    def compute_src2(q, j_row):
        for k in range(CHUNK // L):
            sl = pl.ds(k * L, L)
            src2[q][sl] = src_big[j_row, sl] * 2 + c

    def issue_gather(q):
        pltpu.async_copy(tab32.at[src2[q]], rows[q], gsem[q])

    def wait_gather(q):
        pltpu.make_async_copy(tab32.at[src2[q]], rows[q], gsem[q]).wait()

    def scale_and_scatter(p, j_row):
        for g in range(CHUNK // L):
            vv16 = val_big[j_row, pl.ds(g * L, L)]
            for e in range(L):
                r = g * L + e
                vv = jnp.full((L,), vv16[e], jnp.float32)
                rows[p][r, pl.ds(0, L)] = rows[p][r, pl.ds(0, L)] * vv
                rows[p][r, pl.ds(L, L)] = rows[p][r, pl.ds(L, L)] * vv
        for k in range(CHUNK // L):
            sl = pl.ds(k * L, L)
            dst2[p][sl] = dst_big[j_row, sl]
        pltpu.async_copy(rows[p], acc.at[dst2[p]], ssem[p], add=True)

    def wait_scatter(p):
        pltpu.make_async_copy(rows[p], acc.at[dst2[p]], ssem[p]).wait()

    issue_load(0, 0)
    issue_load(0, 1)
    wait_load(0)
    compute_src2(0, 0)
    issue_gather(0)

    def pair_body(t, carry):
        wait_gather(0)
        scale_and_scatter(0, 0)

        @pl.when(t > 0)
        def _():
            wait_scatter(1)
        compute_src2(1, 1)
        issue_gather(1)

        wait_gather(1)
        scale_and_scatter(1, 1)
        issue_load(t + 1, 0)
        wait_load(1)
        wait_scatter(0)
        compute_src2(0, 2)
        issue_gather(0)

        wait_gather(0)
        scale_and_scatter(0, 2)
        wait_scatter(1)
        compute_src2(1, 3)
        issue_gather(1)

        wait_gather(1)
        scale_and_scatter(1, 3)
        issue_load(t + 1, 1)
        wait_load(0)
        wait_scatter(0)
        compute_src2(0, 0)
        issue_gather(0)
        return carry

    lax.fori_loop(0, N_GROUPS, pair_body, 0)

    wait_gather(0)
    wait_scatter(1)
    wait_load(1)
    plsc.subcore_barrier()

    ro = s * ROWS_PER_TILE

    @pl.when(s < NS - 1)
    def _():
        pltpu.sync_copy(acc.at[pl.ds(ro, ROWS_PER_TILE)],
                        raw_h.at[pl.ds(c * N_NODES + ro, ROWS_PER_TILE)])

    @pl.when(s == NS - 1)
    def _():
        ro_l = (NS - 1) * ROWS_PER_TILE
        pltpu.sync_copy(acc.at[pl.ds(ro_l, ROWS_LAST)],
                        raw_h.at[pl.ds(c * N_NODES + ro_l, ROWS_LAST)])


_layer_call = pl.kernel(
    _layer_body,
    out_type=jax.ShapeDtypeStruct((NC * N_NODES, DH), jnp.float32),
    mesh=_mesh,
    compiler_params=_sc_params,
    scratch_types=[
        pltpu.VMEM_SHARED((N_NODES, DH), jnp.float32),
        pltpu.VMEM((UNROLL, CHUNK), jnp.int32),
        pltpu.VMEM((UNROLL, CHUNK), jnp.int32),
        pltpu.VMEM((UNROLL, CHUNK), jnp.float32),
        pltpu.VMEM((CHUNK,), jnp.int32),
        pltpu.VMEM((CHUNK,), jnp.int32),
        pltpu.VMEM((CHUNK,), jnp.int32),
        pltpu.VMEM((CHUNK,), jnp.int32),
        pltpu.VMEM((CHUNK, DH), jnp.float32),
        pltpu.VMEM((CHUNK, DH), jnp.float32),
        pltpu.SemaphoreType.DMA,
        pltpu.SemaphoreType.DMA,
        pltpu.SemaphoreType.DMA,
        pltpu.SemaphoreType.DMA,
        pltpu.SemaphoreType.DMA,
        pltpu.SemaphoreType.DMA,
    ],
)


def _norm_body(scale, raw_ref, accin_ref, norm_ref, accout_ref):
    ra = raw_ref[0]
    rb = raw_ref[1]
    ss = (jnp.sum(ra * ra, axis=1, keepdims=True)
          + jnp.sum(rb * rb, axis=1, keepdims=True))
    r = lax.rsqrt(jnp.maximum(ss, 1e-12))
    full = jnp.concatenate([ra * r, rb * r], axis=1)
    norm_ref[...] = full
    accout_ref[...] = scale * accin_ref[...] + 0.25 * full


_NORM_ROWS = 1000


def _make_norm_call(scale):
    return pl.pallas_call(
        functools.partial(_norm_body, scale),
        grid=(N_NODES // _NORM_ROWS,),
        in_specs=[
            pl.BlockSpec((NC, _NORM_ROWS, DH), lambda i: (0, i, 0)),
            pl.BlockSpec((_NORM_ROWS, D), lambda i: (i, 0)),
        ],
        out_specs=[
            pl.BlockSpec((_NORM_ROWS, D), lambda i: (i, 0)),
            pl.BlockSpec((_NORM_ROWS, D), lambda i: (i, 0)),
        ],
        out_shape=[
            jax.ShapeDtypeStruct((N_NODES, D), jnp.float32),
            jax.ShapeDtypeStruct((N_NODES, D), jnp.float32),
        ],
    )


_norm_first = _make_norm_call(0.25)
_norm_rest = _make_norm_call(1.0)


_B_PER_TILE = BATCH_B // (NC * NS)


def _gather_body(light, users_h, pos_h, neg_h, u_out, p_out, n_out,
                 u_idx, p_idx, n_idx, rows_u, rows_p, rows_n, sem):
    c = lax.axis_index("c")
    s = lax.axis_index("s")
    wid = s * NC + c
    base = wid * _B_PER_TILE

    pltpu.sync_copy(users_h.at[pl.ds(base, _B_PER_TILE)], u_idx)
    pltpu.sync_copy(pos_h.at[pl.ds(base, _B_PER_TILE)], p_idx)
    pltpu.sync_copy(neg_h.at[pl.ds(base, _B_PER_TILE)], n_idx)

    for k in range(_B_PER_TILE // L):
        sl = pl.ds(k * L, L)
        p_idx[sl] = p_idx[sl] + N_USERS
        n_idx[sl] = n_idx[sl] + N_USERS

    pltpu.async_copy(light.at[u_idx], rows_u, sem).wait()
    pltpu.async_copy(light.at[p_idx], rows_p, sem).wait()
    pltpu.async_copy(light.at[n_idx], rows_n, sem).wait()

    pltpu.sync_copy(rows_u, u_out.at[pl.ds(base, _B_PER_TILE)])
    pltpu.sync_copy(rows_p, p_out.at[pl.ds(base, _B_PER_TILE)])
    pltpu.sync_copy(rows_n, n_out.at[pl.ds(base, _B_PER_TILE)])


_gather_call = pl.kernel(
    _gather_body,
    out_type=[
        jax.ShapeDtypeStruct((BATCH_B, D), jnp.float32),
        jax.ShapeDtypeStruct((BATCH_B, D), jnp.float32),
        jax.ShapeDtypeStruct((BATCH_B, D), jnp.float32),
    ],
    mesh=_mesh,
    compiler_params=_sc_params,
    scratch_types=[
        pltpu.VMEM((_B_PER_TILE,), jnp.int32),
        pltpu.VMEM((_B_PER_TILE,), jnp.int32),
        pltpu.VMEM((_B_PER_TILE,), jnp.int32),
        pltpu.VMEM((_B_PER_TILE, D), jnp.float32),
        pltpu.VMEM((_B_PER_TILE, D), jnp.float32),
        pltpu.VMEM((_B_PER_TILE, D), jnp.float32),
        pltpu.SemaphoreType.DMA,
    ],
)


def _dot_body(u_ref, p_ref, n_ref, ps_ref, ns_ref):
    u = u_ref[...]
    ps_ref[...] = jnp.sum(u * p_ref[...], axis=1)
    ns_ref[...] = jnp.sum(u * n_ref[...], axis=1)


_dot_call = pl.pallas_call(
    _dot_body,
    out_shape=[
        jax.ShapeDtypeStruct((BATCH_B,), jnp.float32),
        jax.ShapeDtypeStruct((BATCH_B,), jnp.float32),
    ],
)


def kernel(users, pos_items, neg_items, adj_indices, adj_values, user_table,
           item_table):
    users = users.astype(jnp.int32)
    pos_items = pos_items.astype(jnp.int32)
    neg_items = neg_items.astype(jnp.int32)

    dst = adj_indices[0].astype(jnp.int32)
    src = adj_indices[1].astype(jnp.int32)
    vals = adj_values.astype(jnp.float32)

    pad = E_ALLOC - E
    src_p = jnp.concatenate([src, jnp.zeros((pad,), jnp.int32)]) \
        .reshape(E_ALLOC // CHUNK, CHUNK)
    dst_p = jnp.concatenate([dst, jnp.zeros((pad,), jnp.int32)]) \
        .reshape(E_ALLOC // CHUNK, CHUNK)
    val_p = jnp.concatenate([vals, jnp.zeros((pad,), jnp.float32)]) \
        .reshape(E_ALLOC // CHUNK, CHUNK)

    all_emb = jnp.concatenate([user_table, item_table], axis=0)
    zeros = jnp.zeros((ROWS_PER_TILE, DH), jnp.float32)

    tab = all_emb
    acc = all_emb
    for layer in range(N_LAYERS):
        raw = _layer_call(tab.reshape(NC * N_NODES, DH), src_p, dst_p, val_p,
                          zeros)
        norm_call = _norm_first if layer == 0 else _norm_rest
        tab, acc = norm_call(raw.reshape(NC, N_NODES, DH), acc)

    u_rows, p_rows, n_rows = _gather_call(acc, users, pos_items, neg_items)
    pos_scores, neg_scores = _dot_call(u_rows, p_rows, n_rows)
    return (pos_scores, neg_scores, acc[:N_USERS], acc[N_USERS:])


# speedup vs baseline: 4.9386x; 1.1484x over previous
import functools

import jax
import jax.numpy as jnp
from jax import lax
from jax.experimental import pallas as pl
from jax.experimental.pallas import tpu as pltpu
from jax.experimental.pallas import tpu_sc as plsc

NC = 2
NS = 16
L = 16

N_USERS = 25000
N_ITEMS = 25000
N_NODES = N_USERS + N_ITEMS
D = 64
DH = D // 2
N_LAYERS = 3
BATCH_B = 4096

E = 800000
CHUNK = 128
UNROLL = 4
N_GROUPS = 100
CHUNKS_PER_TILE = UNROLL * N_GROUPS
E_PAD = NS * CHUNKS_PER_TILE * CHUNK
E_ALLOC = E_PAD + 8 * CHUNK
ROWS_PER_TILE = 3128
ROWS_LAST = N_NODES - (NS - 1) * ROWS_PER_TILE

_mesh = plsc.VectorSubcoreMesh(core_axis_name="c", subcore_axis_name="s")
_sc_params = pltpu.CompilerParams(use_tc_tiling_on_sc=False)


def _layer_body(tab32, src_h, dst_h, val_h, zeros_h, raw_h,
                acc, src_big, dst_big, val_big,
                src2_0, src2_1, src2_2, src2_3,
                dst2_0, dst2_1, dst2_2, dst2_3,
                rows_0, rows_1, rows_2, rows_3,
                gsem_0, gsem_1, gsem_2, gsem_3,
                ssem_0, ssem_1, ssem_2, ssem_3, isem):
    c = lax.axis_index("c")
    s = lax.axis_index("s")

    @pl.when(s < NS - 1)
    def _():
        pltpu.sync_copy(zeros_h,
                        acc.at[pl.ds(s * ROWS_PER_TILE, ROWS_PER_TILE)])

    @pl.when(s == NS - 1)
    def _():
        pltpu.sync_copy(zeros_h.at[pl.ds(0, ROWS_LAST)],
                        acc.at[pl.ds((NS - 1) * ROWS_PER_TILE, ROWS_LAST)])

    plsc.subcore_barrier()

    tile_row0 = s * CHUNKS_PER_TILE
    src2 = (src2_0, src2_1, src2_2, src2_3)
    dst2 = (dst2_0, dst2_1, dst2_2, dst2_3)
    rows = (rows_0, rows_1, rows_2, rows_3)
    gsem = (gsem_0, gsem_1, gsem_2, gsem_3)
    ssem = (ssem_0, ssem_1, ssem_2, ssem_3)

    def issue_load(hrow, brow):
        for h, b in ((src_h, src_big), (dst_h, dst_big), (val_h, val_big)):
            pltpu.async_copy(h.at[pl.ds(hrow, UNROLL), :],
                             b.at[pl.ds(brow, UNROLL), :], isem)

    def wait_load():
        for h, b in ((src_h, src_big), (dst_h, dst_big), (val_h, val_big)):
            pltpu.make_async_copy(h.at[pl.ds(0, UNROLL), :],
                                  b.at[pl.ds(0, UNROLL), :], isem).wait()

    def compute_src2(q, row):
        for k in range(CHUNK // L):
            sl = pl.ds(k * L, L)
            src2[q][sl] = src_big[row, sl] * 2 + c

    def issue_gather(q):
        pltpu.async_copy(tab32.at[src2[q]], rows[q], gsem[q])

    def wait_gather(q):
        pltpu.make_async_copy(tab32.at[src2[q]], rows[q], gsem[q]).wait()

    def scale_and_scatter(p, row):
        for g in range(CHUNK // L):
            vv16 = val_big[row, pl.ds(g * L, L)]
            for e in range(L):
                r = g * L + e
                vv = jnp.full((L,), vv16[e], jnp.float32)
                rows[p][r, pl.ds(0, L)] = rows[p][r, pl.ds(0, L)] * vv
                rows[p][r, pl.ds(L, L)] = rows[p][r, pl.ds(L, L)] * vv
        for k in range(CHUNK // L):
            sl = pl.ds(k * L, L)
            dst2[p][sl] = dst_big[row, sl]
        pltpu.async_copy(rows[p], acc.at[dst2[p]], ssem[p], add=True)

    def wait_scatter(p):
        pltpu.make_async_copy(rows[p], acc.at[dst2[p]], ssem[p]).wait()

    issue_load(tile_row0, 0)
    wait_load()
    for j in range(3):
        compute_src2(j, j)
        issue_gather(j)

    def body(t, carry):
        h0 = (t % 2) * UNROLL
        h1 = ((t + 1) % 2) * UNROLL
        issue_load(tile_row0 + (t + 1) * UNROLL, h1)
        for j in range(UNROLL):
            sj = (j + 3) % 4
            wait_gather(j)
            scale_and_scatter(j, h0 + j)
            if j == 1:
                wait_load()

            if j == 0:
                @pl.when(t > 0)
                def _():
                    wait_scatter(sj)
            else:
                wait_scatter(sj)
            if j == 0:
                compute_src2(sj, h0 + 3)
            else:
                compute_src2(sj, h1 + j - 1)
            issue_gather(sj)
        return carry

    lax.fori_loop(0, N_GROUPS, body, 0)

    wait_gather(0)
    wait_gather(1)
    wait_gather(2)
    wait_scatter(3)
    plsc.subcore_barrier()

    ro = s * ROWS_PER_TILE

    @pl.when(s < NS - 1)
    def _():
        pltpu.sync_copy(acc.at[pl.ds(ro, ROWS_PER_TILE)],
                        raw_h.at[pl.ds(c * N_NODES + ro, ROWS_PER_TILE)])

    @pl.when(s == NS - 1)
    def _():
        ro_l = (NS - 1) * ROWS_PER_TILE
        pltpu.sync_copy(acc.at[pl.ds(ro_l, ROWS_LAST)],
                        raw_h.at[pl.ds(c * N_NODES + ro_l, ROWS_LAST)])


_layer_call = pl.kernel(
    _layer_body,
    out_type=jax.ShapeDtypeStruct((NC * N_NODES, DH), jnp.float32),
    mesh=_mesh,
    compiler_params=_sc_params,
    scratch_types=[
        pltpu.VMEM_SHARED((N_NODES, DH), jnp.float32),
        pltpu.VMEM((2 * UNROLL, CHUNK), jnp.int32),
        pltpu.VMEM((2 * UNROLL, CHUNK), jnp.int32),
        pltpu.VMEM((2 * UNROLL, CHUNK), jnp.float32),
        pltpu.VMEM((CHUNK,), jnp.int32),
        pltpu.VMEM((CHUNK,), jnp.int32),
        pltpu.VMEM((CHUNK,), jnp.int32),
        pltpu.VMEM((CHUNK,), jnp.int32),
        pltpu.VMEM((CHUNK,), jnp.int32),
        pltpu.VMEM((CHUNK,), jnp.int32),
        pltpu.VMEM((CHUNK,), jnp.int32),
        pltpu.VMEM((CHUNK,), jnp.int32),
        pltpu.VMEM((CHUNK, DH), jnp.float32),
        pltpu.VMEM((CHUNK, DH), jnp.float32),
        pltpu.VMEM((CHUNK, DH), jnp.float32),
        pltpu.VMEM((CHUNK, DH), jnp.float32),
        pltpu.SemaphoreType.DMA,
        pltpu.SemaphoreType.DMA,
        pltpu.SemaphoreType.DMA,
        pltpu.SemaphoreType.DMA,
        pltpu.SemaphoreType.DMA,
        pltpu.SemaphoreType.DMA,
        pltpu.SemaphoreType.DMA,
        pltpu.SemaphoreType.DMA,
        pltpu.SemaphoreType.DMA,
    ],
)


def _norm_body(scale, raw_ref, accin_ref, norm_ref, accout_ref):
    ra = raw_ref[0]
    rb = raw_ref[1]
    ss = (jnp.sum(ra * ra, axis=1, keepdims=True)
          + jnp.sum(rb * rb, axis=1, keepdims=True))
    r = lax.rsqrt(jnp.maximum(ss, 1e-12))
    full = jnp.concatenate([ra * r, rb * r], axis=1)
    norm_ref[...] = full
    accout_ref[...] = scale * accin_ref[...] + 0.25 * full


_NORM_ROWS = 1000


def _make_norm_call(scale):
    return pl.pallas_call(
        functools.partial(_norm_body, scale),
        grid=(N_NODES // _NORM_ROWS,),
        in_specs=[
            pl.BlockSpec((NC, _NORM_ROWS, DH), lambda i: (0, i, 0)),
            pl.BlockSpec((_NORM_ROWS, D), lambda i: (i, 0)),
        ],
        out_specs=[
            pl.BlockSpec((_NORM_ROWS, D), lambda i: (i, 0)),
            pl.BlockSpec((_NORM_ROWS, D), lambda i: (i, 0)),
        ],
        out_shape=[
            jax.ShapeDtypeStruct((N_NODES, D), jnp.float32),
            jax.ShapeDtypeStruct((N_NODES, D), jnp.float32),
        ],
    )


_norm_first = _make_norm_call(0.25)
_norm_rest = _make_norm_call(1.0)


_B_PER_TILE = BATCH_B // (NC * NS)


def _gather_body(light, users_h, pos_h, neg_h, u_out, p_out, n_out,
                 u_idx, p_idx, n_idx, rows_u, rows_p, rows_n, sem):
    c = lax.axis_index("c")
    s = lax.axis_index("s")
    wid = s * NC + c
    base = wid * _B_PER_TILE

    pltpu.sync_copy(users_h.at[pl.ds(base, _B_PER_TILE)], u_idx)
    pltpu.sync_copy(pos_h.at[pl.ds(base, _B_PER_TILE)], p_idx)
    pltpu.sync_copy(neg_h.at[pl.ds(base, _B_PER_TILE)], n_idx)

    for k in range(_B_PER_TILE // L):
        sl = pl.ds(k * L, L)
        p_idx[sl] = p_idx[sl] + N_USERS
        n_idx[sl] = n_idx[sl] + N_USERS

    pltpu.async_copy(light.at[u_idx], rows_u, sem).wait()
    pltpu.async_copy(light.at[p_idx], rows_p, sem).wait()
    pltpu.async_copy(light.at[n_idx], rows_n, sem).wait()

    pltpu.sync_copy(rows_u, u_out.at[pl.ds(base, _B_PER_TILE)])
    pltpu.sync_copy(rows_p, p_out.at[pl.ds(base, _B_PER_TILE)])
    pltpu.sync_copy(rows_n, n_out.at[pl.ds(base, _B_PER_TILE)])


_gather_call = pl.kernel(
    _gather_body,
    out_type=[
        jax.ShapeDtypeStruct((BATCH_B, D), jnp.float32),
        jax.ShapeDtypeStruct((BATCH_B, D), jnp.float32),
        jax.ShapeDtypeStruct((BATCH_B, D), jnp.float32),
    ],
    mesh=_mesh,
    compiler_params=_sc_params,
    scratch_types=[
        pltpu.VMEM((_B_PER_TILE,), jnp.int32),
        pltpu.VMEM((_B_PER_TILE,), jnp.int32),
        pltpu.VMEM((_B_PER_TILE,), jnp.int32),
        pltpu.VMEM((_B_PER_TILE, D), jnp.float32),
        pltpu.VMEM((_B_PER_TILE, D), jnp.float32),
        pltpu.VMEM((_B_PER_TILE, D), jnp.float32),
        pltpu.SemaphoreType.DMA,
    ],
)


def _dot_body(u_ref, p_ref, n_ref, ps_ref, ns_ref):
    u = u_ref[...]
    ps_ref[...] = jnp.sum(u * p_ref[...], axis=1)
    ns_ref[...] = jnp.sum(u * n_ref[...], axis=1)


_dot_call = pl.pallas_call(
    _dot_body,
    out_shape=[
        jax.ShapeDtypeStruct((BATCH_B,), jnp.float32),
        jax.ShapeDtypeStruct((BATCH_B,), jnp.float32),
    ],
)


def kernel(users, pos_items, neg_items, adj_indices, adj_values, user_table,
           item_table):
    users = users.astype(jnp.int32)
    pos_items = pos_items.astype(jnp.int32)
    neg_items = neg_items.astype(jnp.int32)

    dst = adj_indices[0].astype(jnp.int32)
    src = adj_indices[1].astype(jnp.int32)
    vals = adj_values.astype(jnp.float32)

    pad = E_ALLOC - E
    src_p = jnp.concatenate([src, jnp.zeros((pad,), jnp.int32)]) \
        .reshape(E_ALLOC // CHUNK, CHUNK)
    dst_p = jnp.concatenate([dst, jnp.zeros((pad,), jnp.int32)]) \
        .reshape(E_ALLOC // CHUNK, CHUNK)
    val_p = jnp.concatenate([vals, jnp.zeros((pad,), jnp.float32)]) \
        .reshape(E_ALLOC // CHUNK, CHUNK)

    all_emb = jnp.concatenate([user_table, item_table], axis=0)
    zeros = jnp.zeros((ROWS_PER_TILE, DH), jnp.float32)

    tab = all_emb
    acc = all_emb
    for layer in range(N_LAYERS):
        raw = _layer_call(tab.reshape(NC * N_NODES, DH), src_p, dst_p, val_p,
                          zeros)
        norm_call = _norm_first if layer == 0 else _norm_rest
        tab, acc = norm_call(raw.reshape(NC, N_NODES, DH), acc)

    u_rows, p_rows, n_rows = _gather_call(acc, users, pos_items, neg_items)
    pos_scores, neg_scores = _dot_call(u_rows, p_rows, n_rows)
    return (pos_scores, neg_scores, acc[:N_USERS], acc[N_USERS:])
